# baseline (speedup 1.0000x reference)
import functools

import jax
import jax.numpy as jnp
from jax import lax
from jax.experimental import pallas as pl
from jax.experimental.pallas import tpu as pltpu
from jax.experimental.pallas import tpu_sc as plsc

NN = 10000
EE = 320000
HH = 128
NLAYER = 4

NP = 10240
NC = 2
NS = 16
NW = NC * NS
CHUNK = 128
NCHUNK = 80
EPW = CHUNK * NCHUNK
EP = NW * EPW
ROWS_PER_TILE = NP // NS
GW = HH + 16

_f32 = jnp.float32


def _silu(x):
    return x * jax.nn.sigmoid(x)


def _sc_mesh():
    return plsc.VectorSubcoreMesh(
        core_axis_name="c", subcore_axis_name="s", num_cores=NC, num_subcores=NS)


def _gather_body(a_h, b_h, dst2_h, src2_h,
                 gd_h, gs_h,
                 di2, si2, ga0, ga1, gb0, gb1,
                 semg0, semg1, semw0, semw1):
    cid = lax.axis_index("c")
    sid = lax.axis_index("s")
    wid = sid * NC + cid
    base = wid * EPW
    ga = (ga0, ga1)
    gb = (gb0, gb1)
    semg = (semg0, semg1)
    semw = (semw0, semw1)

    pltpu.sync_copy(dst2_h.at[pl.ds(wid * NCHUNK, NCHUNK)], di2)
    pltpu.sync_copy(src2_h.at[pl.ds(wid * NCHUNK, NCHUNK)], si2)

    def outer(g, carry):
        for b in range(2):
            ci = g * 2 + b
            @pl.when(ci >= 2)
            def _():
                pltpu.make_async_copy(ga[b], gd_h.at[pl.ds(0, CHUNK)],
                                      semw[b]).wait()
                pltpu.make_async_copy(gb[b], gs_h.at[pl.ds(0, CHUNK)],
                                      semw[b]).wait()
            pltpu.async_copy(a_h.at[di2.at[ci]], ga[b], semg[b])
            pltpu.async_copy(b_h.at[si2.at[ci]], gb[b], semg[b])

            @pl.when(ci >= 1)
            def _():
                o = base + (ci - 1) * CHUNK
                pltpu.make_async_copy(a_h.at[di2.at[ci]], ga[1 - b],
                                      semg[1 - b]).wait()
                pltpu.make_async_copy(b_h.at[si2.at[ci]], gb[1 - b],
                                      semg[1 - b]).wait()
                pltpu.async_copy(ga[1 - b], gd_h.at[pl.ds(o, CHUNK)],
                                 semw[1 - b])
                pltpu.async_copy(gb[1 - b], gs_h.at[pl.ds(o, CHUNK)],
                                 semw[1 - b])
        return carry

    lax.fori_loop(0, NCHUNK // 2, outer, 0)
    last = NCHUNK - 1
    o = base + last * CHUNK
    pltpu.make_async_copy(ga[1], gd_h.at[pl.ds(o, CHUNK)], semg[1]).wait()
    pltpu.make_async_copy(gb[1], gs_h.at[pl.ds(o, CHUNK)], semg[1]).wait()
    pltpu.async_copy(ga[1], gd_h.at[pl.ds(o, CHUNK)], semw[1])
    pltpu.async_copy(gb[1], gs_h.at[pl.ds(o, CHUNK)], semw[1])
    for b in range(2):
        pltpu.make_async_copy(ga[b], gd_h.at[pl.ds(0, CHUNK)], semw[b]).wait()
        pltpu.make_async_copy(gb[b], gs_h.at[pl.ds(0, CHUNK)], semw[b]).wait()


def _sc_gather(A, B, dst2, src2):
    out_type = (jax.ShapeDtypeStruct((EP, GW), _f32),
                jax.ShapeDtypeStruct((EP, GW), _f32))
    scratch = [pltpu.VMEM((NCHUNK, CHUNK), jnp.int32),
               pltpu.VMEM((NCHUNK, CHUNK), jnp.int32),
               pltpu.VMEM((CHUNK, GW), _f32),
               pltpu.VMEM((CHUNK, GW), _f32),
               pltpu.VMEM((CHUNK, GW), _f32),
               pltpu.VMEM((CHUNK, GW), _f32),
               pltpu.SemaphoreType.DMA,
               pltpu.SemaphoreType.DMA,
               pltpu.SemaphoreType.DMA,
               pltpu.SemaphoreType.DMA]
    fn = pl.kernel(_gather_body, out_type=out_type, mesh=_sc_mesh(),
                   scratch_types=scratch, name="sc_gather",
                   compiler_params=pltpu.CompilerParams(
                       use_tc_tiling_on_sc=False))
    return fn(A, B, dst2, src2)


def _zero_vmem(ref, nrow, ncol):
    z = jnp.zeros((16,), _f32)

    def row(r, carry):
        for j in range(ncol // 16):
            ref[r, pl.ds(j * 16, 16)] = z
        return carry

    lax.fori_loop(0, nrow, row, 0)


def _scatter_body(has_t, *refs):
    if has_t:
        (m2_h, t_h, dst_h, magg_h, tacc_h,
         di0, di1, mv0, mv1, tv0, tv1, accm, acct,
         seml0, seml1, sema0, sema1) = refs
    else:
        (m2_h, dst_h, magg_h,
         di0, di1, mv0, mv1, accm,
         seml0, seml1, sema0, sema1) = refs
    cid = lax.axis_index("c")
    sid = lax.axis_index("s")
    wid = sid * NC + cid
    base = wid * EPW
    row0 = sid * ROWS_PER_TILE
    di = (di0, di1)
    mv = (mv0, mv1)
    tv = (tv0, tv1) if has_t else None
    seml = (seml0, seml1)
    sema = (sema0, sema1)

    _zero_vmem(mv0, CHUNK, HH)
    if has_t:
        _zero_vmem(tv0, CHUNK, 16)
    for j in range(ROWS_PER_TILE // CHUNK):
        pltpu.sync_copy(mv0, accm.at[pl.ds(row0 + j * CHUNK, CHUNK)])
        if has_t:
            pltpu.sync_copy(tv0, acct.at[pl.ds(row0 + j * CHUNK, CHUNK)])
    plsc.subcore_barrier()

    def loads(ci, b):
        o = base + ci * CHUNK
        pltpu.async_copy(dst_h.at[pl.ds(o, CHUNK)], di[b], seml[b])
        pltpu.async_copy(m2_h.at[pl.ds(o, CHUNK)], mv[b], seml[b])
        if has_t:
            pltpu.async_copy(t_h.at[pl.ds(o, CHUNK)], tv[b], seml[b])

    def adds(b):
        pltpu.async_copy(mv[b], accm.at[di[b]], sema[b], add=True)
        if has_t:
            pltpu.async_copy(tv[b], acct.at[di[b]], sema[b], add=True)

    def drain_loads(b):
        pltpu.make_async_copy(dst_h.at[pl.ds(0, CHUNK)], di[b],
                              seml[b]).wait()
        pltpu.make_async_copy(m2_h.at[pl.ds(0, CHUNK)], mv[b],
                              seml[b]).wait()
        if has_t:
            pltpu.make_async_copy(t_h.at[pl.ds(0, CHUNK)], tv[b],
                                  seml[b]).wait()

    def drain_adds(b):
        pltpu.make_async_copy(m2_h.at[pl.ds(0, CHUNK)], mv[b],
                              sema[b]).wait()
        if has_t:
            pltpu.make_async_copy(t_h.at[pl.ds(0, CHUNK)], tv[b],
                                  sema[b]).wait()

    def outer(g, carry):
        for b in range(2):
            ci = g * 2 + b

            @pl.when(ci >= 2)
            def _():
                drain_adds(b)
            loads(ci, b)

            @pl.when(ci >= 1)
            def _():
                drain_loads(1 - b)
                adds(1 - b)
        return carry

    lax.fori_loop(0, NCHUNK // 2, outer, 0)
    drain_loads(1)
    adds(1)
    for b in range(2):
        drain_adds(b)
    plsc.subcore_barrier()

    for j in range(ROWS_PER_TILE // CHUNK):
        r = row0 + j * CHUNK
        pltpu.sync_copy(accm.at[pl.ds(r, CHUNK)], magg_h.at[cid, pl.ds(r, CHUNK)])
        if has_t:
            pltpu.sync_copy(acct.at[pl.ds(r, CHUNK)],
                            tacc_h.at[cid, pl.ds(r, CHUNK)])


def _sc_scatter(m2, T, dstp):
    has_t = T is not None
    sems = [pltpu.SemaphoreType.DMA] * 4
    idx = [pltpu.VMEM((CHUNK,), jnp.int32), pltpu.VMEM((CHUNK,), jnp.int32)]
    if has_t:
        out_type = (jax.ShapeDtypeStruct((NC, NP, HH), _f32),
                    jax.ShapeDtypeStruct((NC, NP, 16), _f32))
        scratch = idx + [pltpu.VMEM((CHUNK, HH), _f32),
                         pltpu.VMEM((CHUNK, HH), _f32),
                         pltpu.VMEM((CHUNK, 16), _f32),
                         pltpu.VMEM((CHUNK, 16), _f32),
                         pltpu.VMEM_SHARED((NP, HH), _f32),
                         pltpu.VMEM_SHARED((NP, 16), _f32)] + sems
    else:
        out_type = jax.ShapeDtypeStruct((NC, NP, HH), _f32)
        scratch = idx + [pltpu.VMEM((CHUNK, HH), _f32),
                         pltpu.VMEM((CHUNK, HH), _f32),
                         pltpu.VMEM_SHARED((NP, HH), _f32)] + sems
    fn = pl.kernel(functools.partial(_scatter_body, has_t),
                   out_type=out_type, mesh=_sc_mesh(), scratch_types=scratch,
                   name="sc_scatter",
                   compiler_params=pltpu.CompilerParams(
                       use_tc_tiling_on_sc=False))
    args = (m2, T, dstp) if has_t else (m2, dstp)
    return fn(*args)


BE = 512
BN = 512


def _full(x):
    return pl.BlockSpec(x.shape, lambda i: (0,) * x.ndim)


def _blk(bs):
    nd = len(bs)
    return pl.BlockSpec(bs, lambda i: (i,) + (0,) * (nd - 1))


def _edge_kernel_body(coord, gdm, gsm, ea, we, w1e, b1, be_, wd2,
                      w2, b2, c1, c1b, c2r, m2_o, t_o=None):
    i = pl.program_id(0)
    m_blk = jnp.dot(we[...], w1e[...], preferred_element_type=_f32)
    b1p = b1[...] + jnp.dot(be_[...], w1e[...], preferred_element_type=_f32)
    gd = gdm[...][:, :HH]
    gs = gsm[...][:, :HH]
    rel = gdm[...][:, HH:] - gsm[...][:, HH:]
    d2 = jnp.sum(rel * rel, axis=1, keepdims=True)
    pre = (gd + gs + d2 * wd2[...]
           + jnp.dot(ea[...], m_blk, preferred_element_type=_f32) + b1p)
    m = _silu(pre)
    m2 = _silu(jnp.dot(m, w2[...], preferred_element_type=_f32) + b2[...])
    eid = i * BE + lax.broadcasted_iota(jnp.int32, (BE, 1), 0)
    valid = eid < EE
    m2 = jnp.where(valid, m2, 0.0)
    m2_o[...] = m2
    if coord:
        u2 = _silu(jnp.dot(m2, c1[...], preferred_element_type=_f32) + c1b[...])
        cw = jnp.sum(u2 * c2r[...], axis=1, keepdims=True)
        lane3 = lax.broadcasted_iota(jnp.int32, (1, 16), 1) == 3
        t = rel * cw + lane3.astype(_f32)
        t_o[...] = jnp.where(valid, t, 0.0)


def _tc_edge(coord, gdm, gsm, eap, we, w1e, b1, be_, wd2, w2, b2,
             c1, c1b, c2r):
    grid = EP // BE
    in_specs = [_blk((BE, GW)), _blk((BE, GW)),
                _blk((BE, 16))] + [_full(w) for w in
                                   (we, w1e, b1, be_, wd2, w2, b2, c1, c1b, c2r)]
    if coord:
        out_shape = (jax.ShapeDtypeStruct((EP, HH), _f32),
                     jax.ShapeDtypeStruct((EP, 16), _f32))
        out_specs = (_blk((BE, HH)), _blk((BE, 16)))
    else:
        out_shape = jax.ShapeDtypeStruct((EP, HH), _f32)
        out_specs = _blk((BE, HH))
    return pl.pallas_call(
        functools.partial(_edge_kernel_body, coord),
        grid=(grid,), in_specs=in_specs, out_specs=out_specs,
        out_shape=out_shape)(gdm, gsm, eap, we, w1e, b1, be_, wd2,
                             w2, b2, c1, c1b, c2r)


def _node_kernel_body(coord, h, m0, m1, t0, t1, xq, n1a, n1b, b1n, n2, b2n,
                      g, bb, w1d, w1s, h_o, xq_o=None, a_o=None, b_o=None):
    hv = h[...]
    magg = m0[...] + m1[...]
    u = _silu(jnp.dot(hv, n1a[...], preferred_element_type=_f32)
              + jnp.dot(magg, n1b[...], preferred_element_type=_f32) + b1n[...])
    hn = hv + jnp.dot(u, n2[...], preferred_element_type=_f32) + b2n[...]
    mu = jnp.mean(hn, axis=1, keepdims=True)
    ctr = hn - mu
    var = jnp.mean(ctr * ctr, axis=1, keepdims=True)
    hln = ctr * jax.lax.rsqrt(var + 1e-5) * g[...] + bb[...]
    h_o[...] = hln
    if coord:
        tacc = t0[...] + t1[...]
        deg = tacc[:, 3:4]
        invd = 1.0 / jnp.maximum(deg, 1.0)
        lane = lax.broadcasted_iota(jnp.int32, (1, 16), 1)
        xq_n = xq[...] + jnp.where(lane < 3, tacc, 0.0) * invd
        xq_o[...] = xq_n
        a_o[...] = jnp.concatenate(
            [jnp.dot(hln, w1d[...], preferred_element_type=_f32), xq_n], axis=1)
        b_o[...] = jnp.concatenate(
            [jnp.dot(hln, w1s[...], preferred_element_type=_f32), xq_n], axis=1)


def _tc_node(coord, h, m0, m1, t0, t1, xq, n1a, n1b, b1n, n2, b2n, g, bb,
             w1d, w1s):
    grid = NP // BN
    in_specs = [_blk((BN, HH)), _blk((BN, HH)), _blk((BN, HH)),
                _blk((BN, 16)), _blk((BN, 16)), _blk((BN, 16))] + \
               [_full(w) for w in (n1a, n1b, b1n, n2, b2n, g, bb, w1d, w1s)]
    if coord:
        out_shape = (jax.ShapeDtypeStruct((NP, HH), _f32),
                     jax.ShapeDtypeStruct((NP, 16), _f32),
                     jax.ShapeDtypeStruct((NP, GW), _f32),
                     jax.ShapeDtypeStruct((NP, GW), _f32))
        out_specs = (_blk((BN, HH)), _blk((BN, 16)), _blk((BN, GW)),
                     _blk((BN, GW)))
    else:
        out_shape = jax.ShapeDtypeStruct((NP, HH), _f32)
        out_specs = _blk((BN, HH))
    return pl.pallas_call(
        functools.partial(_node_kernel_body, coord),
        grid=(grid,), in_specs=in_specs, out_specs=out_specs,
        out_shape=out_shape)(h, m0, m1, t0, t1, xq, n1a, n1b, b1n, n2, b2n,
                             g, bb, w1d, w1s)


def _init_kernel_body(nf, pos, wn, bn, w1d, w1s, h_o, xq_o, a_o, b_o):
    h = (jnp.dot(jnp.clip(nf[...], -100.0, 100.0), wn[...],
                 preferred_element_type=_f32) + bn[...])
    h_o[...] = h
    xq = jnp.clip(pos[...], -500.0, 500.0)
    xq_o[...] = xq
    a_o[...] = jnp.concatenate(
        [jnp.dot(h, w1d[...], preferred_element_type=_f32), xq], axis=1)
    b_o[...] = jnp.concatenate(
        [jnp.dot(h, w1s[...], preferred_element_type=_f32), xq], axis=1)


def _tc_init(nfp, pos16, wn, bn, w1d, w1s):
    grid = NP // BN
    in_specs = [_blk((BN, 128)), _blk((BN, 16))] + \
               [_full(w) for w in (wn, bn, w1d, w1s)]
    out_shape = (jax.ShapeDtypeStruct((NP, HH), _f32),
                 jax.ShapeDtypeStruct((NP, 16), _f32),
                 jax.ShapeDtypeStruct((NP, GW), _f32),
                 jax.ShapeDtypeStruct((NP, GW), _f32))
    out_specs = (_blk((BN, HH)), _blk((BN, 16)), _blk((BN, GW)),
                 _blk((BN, GW)))
    return pl.pallas_call(
        _init_kernel_body, grid=(grid,), in_specs=in_specs,
        out_specs=out_specs, out_shape=out_shape)(nfp, pos16, wn, bn,
                                                  w1d, w1s)


def kernel(node_features, positions, edge_index, edge_attr, params):
    src = edge_index[0]
    dst = edge_index[1]
    dstp = jnp.pad(dst, (0, EP - EE), constant_values=NN)
    dst2 = dstp.reshape(NW * NCHUNK, CHUNK)
    src2 = jnp.pad(src, (0, EP - EE), constant_values=NN).reshape(
        NW * NCHUNK, CHUNK)
    eap = jnp.pad(edge_attr, ((0, EP - EE), (0, 0)))
    nfp = jnp.pad(node_features, ((0, NP - NN), (0, 0)))
    pos16 = jnp.pad(positions, ((0, NP - NN), (0, 13)))

    we = params["edge_embed"]["W"]
    be_ = params["edge_embed"]["b"][None]
    lw = []
    for lp in params["layers"]:
        w1 = lp["edge1"]["W"]
        lw.append(dict(
            w1d=w1[:HH], w1s=w1[HH:2 * HH], wd2=w1[2 * HH:2 * HH + 1],
            w1e=w1[2 * HH + 1:], b1=lp["edge1"]["b"][None],
            w2=lp["edge2"]["W"], b2=lp["edge2"]["b"][None],
            c1=lp["coord1"]["W"], c1b=lp["coord1"]["b"][None],
            c2r=lp["coord2"]["W"].T,
            n1a=lp["node1"]["W"][:HH], n1b=lp["node1"]["W"][HH:],
            b1n=lp["node1"]["b"][None], n2=lp["node2"]["W"],
            b2n=lp["node2"]["b"][None], g=lp["ln_g"][None],
            bb=lp["ln_b"][None]))

    h, xq, A, B = _tc_init(nfp, pos16, params["node_embed"]["W"],
                           params["node_embed"]["b"][None],
                           lw[0]["w1d"], lw[0]["w1s"])

    for i in range(NLAYER):
        w = lw[i]
        coord = i < NLAYER - 1
        GDM, GSM = _sc_gather(A, B, dst2, src2)
        if coord:
            m2, T = _tc_edge(True, GDM, GSM, eap, we, w["w1e"],
                             w["b1"], be_, w["wd2"], w["w2"], w["b2"],
                             w["c1"], w["c1b"], w["c2r"])
            Magg, Tacc = _sc_scatter(m2, T, dstp)
            nx = lw[i + 1]
            h, xq, A, B = _tc_node(
                True, h, Magg[0], Magg[1], Tacc[0], Tacc[1], xq,
                w["n1a"], w["n1b"], w["b1n"], w["n2"], w["b2n"],
                w["g"], w["bb"], nx["w1d"], nx["w1s"])
        else:
            m2 = _tc_edge(False, GDM, GSM, eap, we, w["w1e"],
                          w["b1"], be_, w["wd2"], w["w2"], w["b2"],
                          w["c1"], w["c1b"], w["c2r"])
            Magg = _sc_scatter(m2, None, dstp)
            zt = jnp.zeros((NP, 16), _f32)
            h = _tc_node(False, h, Magg[0], Magg[1], zt, zt, xq,
                         w["n1a"], w["n1b"], w["b1n"], w["n2"], w["b2n"],
                         w["g"], w["bb"], w["w1d"], w["w1s"])
    return h[:NN]


# speedup vs baseline: 1.2439x; 1.2439x over previous
import functools

import jax
import jax.numpy as jnp
from jax import lax
from jax.experimental import pallas as pl
from jax.experimental.pallas import tpu as pltpu
from jax.experimental.pallas import tpu_sc as plsc

NN = 10000
EE = 320000
HH = 128
NLAYER = 4

NP = 10240
NC = 2
NS = 16
NW = NC * NS
CHUNK = 128
NCHUNK = 80
EPW = CHUNK * NCHUNK
EP = NW * EPW
ROWS_PER_TILE = NP // NS
GW = HH + 16

_f32 = jnp.float32


def _silu(x):
    return x * jax.nn.sigmoid(x)


def _sc_mesh():
    return plsc.VectorSubcoreMesh(
        core_axis_name="c", subcore_axis_name="s", num_cores=NC, num_subcores=NS)


def _gather_body(a_h, b_h, xq_h, dst2_h, src2_h,
                 gd_h, gs_h, xd_h, xs_h,
                 di2, si2, a0, a1, b0, b1, xd0, xd1, xs0, xs1,
                 semg0, semg1, semw0, semw1):
    cid = lax.axis_index("c")
    sid = lax.axis_index("s")
    wid = sid * NC + cid
    base = wid * EPW
    av = (a0, a1)
    bv = (b0, b1)
    xdv = (xd0, xd1)
    xsv = (xs0, xs1)
    semg = (semg0, semg1)
    semw = (semw0, semw1)

    pltpu.sync_copy(dst2_h.at[pl.ds(wid * NCHUNK, NCHUNK)], di2)
    pltpu.sync_copy(src2_h.at[pl.ds(wid * NCHUNK, NCHUNK)], si2)

    def gathers(ci, b):
        pltpu.async_copy(a_h.at[di2.at[ci]], av[b], semg[b])
        pltpu.async_copy(b_h.at[si2.at[ci]], bv[b], semg[b])
        pltpu.async_copy(xq_h.at[di2.at[ci]], xdv[b], semg[b])
        pltpu.async_copy(xq_h.at[si2.at[ci]], xsv[b], semg[b])

    def drain(sems, b):
        pltpu.make_async_copy(a_h.at[di2.at[0]], av[b], sems[b]).wait()
        pltpu.make_async_copy(b_h.at[si2.at[0]], bv[b], sems[b]).wait()
        pltpu.make_async_copy(xq_h.at[di2.at[0]], xdv[b], sems[b]).wait()
        pltpu.make_async_copy(xq_h.at[si2.at[0]], xsv[b], sems[b]).wait()

    def writeback(ci, b):
        o = base + ci * CHUNK
        pltpu.async_copy(av[b], gd_h.at[pl.ds(o, CHUNK)], semw[b])
        pltpu.async_copy(bv[b], gs_h.at[pl.ds(o, CHUNK)], semw[b])
        pltpu.async_copy(xdv[b], xd_h.at[pl.ds(o, CHUNK)], semw[b])
        pltpu.async_copy(xsv[b], xs_h.at[pl.ds(o, CHUNK)], semw[b])

    def outer(g, carry):
        for b in range(2):
            ci = g * 2 + b

            @pl.when(ci >= 2)
            def _():
                drain(semw, b)
            gathers(ci, b)

            @pl.when(ci >= 1)
            def _():
                drain(semg, 1 - b)
                writeback(ci - 1, 1 - b)
        return carry

    lax.fori_loop(0, NCHUNK // 2, outer, 0)
    drain(semg, 1)
    writeback(NCHUNK - 1, 1)
    for b in range(2):
        drain(semw, b)


def _sc_gather(A, B, XQ, dst2, src2):
    out_type = (jax.ShapeDtypeStruct((EP, HH), _f32),
                jax.ShapeDtypeStruct((EP, HH), _f32),
                jax.ShapeDtypeStruct((EP, 16), _f32),
                jax.ShapeDtypeStruct((EP, 16), _f32))
    scratch = [pltpu.VMEM((NCHUNK, CHUNK), jnp.int32),
               pltpu.VMEM((NCHUNK, CHUNK), jnp.int32),
               pltpu.VMEM((CHUNK, HH), _f32),
               pltpu.VMEM((CHUNK, HH), _f32),
               pltpu.VMEM((CHUNK, HH), _f32),
               pltpu.VMEM((CHUNK, HH), _f32),
               pltpu.VMEM((CHUNK, 16), _f32),
               pltpu.VMEM((CHUNK, 16), _f32),
               pltpu.VMEM((CHUNK, 16), _f32),
               pltpu.VMEM((CHUNK, 16), _f32),
               pltpu.SemaphoreType.DMA,
               pltpu.SemaphoreType.DMA,
               pltpu.SemaphoreType.DMA,
               pltpu.SemaphoreType.DMA]
    fn = pl.kernel(_gather_body, out_type=out_type, mesh=_sc_mesh(),
                   scratch_types=scratch, name="sc_gather",
                   compiler_params=pltpu.CompilerParams(
                       use_tc_tiling_on_sc=False))
    return fn(A, B, XQ, dst2, src2)


def _zero_vmem(ref, nrow, ncol):
    z = jnp.zeros((16,), _f32)

    def row(r, carry):
        for j in range(ncol // 16):
            ref[r, pl.ds(j * 16, 16)] = z
        return carry

    lax.fori_loop(0, nrow, row, 0)


def _scatter_body(has_t, *refs):
    if has_t:
        (m2_h, t_h, dst_h, magg_h, tacc_h,
         di0, di1, mv0, mv1, tv0, tv1, accm, acct,
         seml0, seml1, sema0, sema1) = refs
    else:
        (m2_h, dst_h, magg_h,
         di0, di1, mv0, mv1, accm,
         seml0, seml1, sema0, sema1) = refs
    cid = lax.axis_index("c")
    sid = lax.axis_index("s")
    wid = sid * NC + cid
    base = wid * EPW
    row0 = sid * ROWS_PER_TILE
    di = (di0, di1)
    mv = (mv0, mv1)
    tv = (tv0, tv1) if has_t else None
    seml = (seml0, seml1)
    sema = (sema0, sema1)

    _zero_vmem(mv0, CHUNK, HH)
    if has_t:
        _zero_vmem(tv0, CHUNK, 16)
    for j in range(ROWS_PER_TILE // CHUNK):
        pltpu.sync_copy(mv0, accm.at[pl.ds(row0 + j * CHUNK, CHUNK)])
        if has_t:
            pltpu.sync_copy(tv0, acct.at[pl.ds(row0 + j * CHUNK, CHUNK)])
    plsc.subcore_barrier()

    def loads(ci, b):
        o = base + ci * CHUNK
        pltpu.async_copy(dst_h.at[pl.ds(o, CHUNK)], di[b], seml[b])
        pltpu.async_copy(m2_h.at[pl.ds(o, CHUNK)], mv[b], seml[b])
        if has_t:
            pltpu.async_copy(t_h.at[pl.ds(o, CHUNK)], tv[b], seml[b])

    def adds(b):
        pltpu.async_copy(mv[b], accm.at[di[b]], sema[b], add=True)
        if has_t:
            pltpu.async_copy(tv[b], acct.at[di[b]], sema[b], add=True)

    def drain_loads(b):
        pltpu.make_async_copy(dst_h.at[pl.ds(0, CHUNK)], di[b],
                              seml[b]).wait()
        pltpu.make_async_copy(m2_h.at[pl.ds(0, CHUNK)], mv[b],
                              seml[b]).wait()
        if has_t:
            pltpu.make_async_copy(t_h.at[pl.ds(0, CHUNK)], tv[b],
                                  seml[b]).wait()

    def drain_adds(b):
        pltpu.make_async_copy(m2_h.at[pl.ds(0, CHUNK)], mv[b],
                              sema[b]).wait()
        if has_t:
            pltpu.make_async_copy(t_h.at[pl.ds(0, CHUNK)], tv[b],
                                  sema[b]).wait()

    def outer(g, carry):
        for b in range(2):
            ci = g * 2 + b

            @pl.when(ci >= 2)
            def _():
                drain_adds(b)
            loads(ci, b)

            @pl.when(ci >= 1)
            def _():
                drain_loads(1 - b)
                adds(1 - b)
        return carry

    lax.fori_loop(0, NCHUNK // 2, outer, 0)
    drain_loads(1)
    adds(1)
    for b in range(2):
        drain_adds(b)
    plsc.subcore_barrier()

    for j in range(ROWS_PER_TILE // CHUNK):
        r = row0 + j * CHUNK
        pltpu.sync_copy(accm.at[pl.ds(r, CHUNK)], magg_h.at[cid, pl.ds(r, CHUNK)])
        if has_t:
            pltpu.sync_copy(acct.at[pl.ds(r, CHUNK)],
                            tacc_h.at[cid, pl.ds(r, CHUNK)])


def _sc_scatter(m2, T, dstp):
    has_t = T is not None
    sems = [pltpu.SemaphoreType.DMA] * 4
    idx = [pltpu.VMEM((CHUNK,), jnp.int32), pltpu.VMEM((CHUNK,), jnp.int32)]
    if has_t:
        out_type = (jax.ShapeDtypeStruct((NC, NP, HH), _f32),
                    jax.ShapeDtypeStruct((NC, NP, 16), _f32))
        scratch = idx + [pltpu.VMEM((CHUNK, HH), _f32),
                         pltpu.VMEM((CHUNK, HH), _f32),
                         pltpu.VMEM((CHUNK, 16), _f32),
                         pltpu.VMEM((CHUNK, 16), _f32),
                         pltpu.VMEM_SHARED((NP, HH), _f32),
                         pltpu.VMEM_SHARED((NP, 16), _f32)] + sems
    else:
        out_type = jax.ShapeDtypeStruct((NC, NP, HH), _f32)
        scratch = idx + [pltpu.VMEM((CHUNK, HH), _f32),
                         pltpu.VMEM((CHUNK, HH), _f32),
                         pltpu.VMEM_SHARED((NP, HH), _f32)] + sems
    fn = pl.kernel(functools.partial(_scatter_body, has_t),
                   out_type=out_type, mesh=_sc_mesh(), scratch_types=scratch,
                   name="sc_scatter",
                   compiler_params=pltpu.CompilerParams(
                       use_tc_tiling_on_sc=False))
    args = (m2, T, dstp) if has_t else (m2, dstp)
    return fn(*args)


BE = 512
BN = 512


def _full(x):
    return pl.BlockSpec(x.shape, lambda i: (0,) * x.ndim)


def _blk(bs):
    nd = len(bs)
    return pl.BlockSpec(bs, lambda i: (i,) + (0,) * (nd - 1))


def _edge_kernel_body(coord, gd, gs, xd, xs, ea, we, w1e, b1, be_, wd2,
                      w2, b2, c1, c1b, c2r, m2_o, t_o=None):
    i = pl.program_id(0)
    m_blk = jnp.dot(we[...], w1e[...], preferred_element_type=_f32)
    b1p = b1[...] + jnp.dot(be_[...], w1e[...], preferred_element_type=_f32)
    rel = xd[...] - xs[...]
    d2 = jnp.sum(rel * rel, axis=1, keepdims=True)
    pre = (gd[...] + gs[...] + d2 * wd2[...]
           + jnp.dot(ea[...], m_blk, preferred_element_type=_f32) + b1p)
    m = _silu(pre)
    m2 = _silu(jnp.dot(m, w2[...], preferred_element_type=_f32) + b2[...])
    eid = i * BE + lax.broadcasted_iota(jnp.int32, (BE, 1), 0)
    valid = eid < EE
    m2 = jnp.where(valid, m2, 0.0)
    m2_o[...] = m2
    if coord:
        u2 = _silu(jnp.dot(m2, c1[...], preferred_element_type=_f32) + c1b[...])
        cw = jnp.sum(u2 * c2r[...], axis=1, keepdims=True)
        lane3 = lax.broadcasted_iota(jnp.int32, (1, 16), 1) == 3
        t = rel * cw + lane3.astype(_f32)
        t_o[...] = jnp.where(valid, t, 0.0)


def _tc_edge(coord, gd, gs, xd, xs, eap, we, w1e, b1, be_, wd2, w2, b2,
             c1, c1b, c2r):
    grid = EP // BE
    in_specs = [_blk((BE, HH)), _blk((BE, HH)), _blk((BE, 16)), _blk((BE, 16)),
                _blk((BE, 16))] + [_full(w) for w in
                                   (we, w1e, b1, be_, wd2, w2, b2, c1, c1b, c2r)]
    if coord:
        out_shape = (jax.ShapeDtypeStruct((EP, HH), _f32),
                     jax.ShapeDtypeStruct((EP, 16), _f32))
        out_specs = (_blk((BE, HH)), _blk((BE, 16)))
    else:
        out_shape = jax.ShapeDtypeStruct((EP, HH), _f32)
        out_specs = _blk((BE, HH))
    return pl.pallas_call(
        functools.partial(_edge_kernel_body, coord),
        grid=(grid,), in_specs=in_specs, out_specs=out_specs,
        out_shape=out_shape)(gd, gs, xd, xs, eap, we, w1e, b1, be_, wd2,
                             w2, b2, c1, c1b, c2r)


def _node_kernel_body(coord, h, m0, m1, t0, t1, xq, n1a, n1b, b1n, n2, b2n,
                      g, bb, w1d, w1s, h_o, xq_o=None, a_o=None, b_o=None):
    hv = h[...]
    magg = m0[...] + m1[...]
    u = _silu(jnp.dot(hv, n1a[...], preferred_element_type=_f32)
              + jnp.dot(magg, n1b[...], preferred_element_type=_f32) + b1n[...])
    hn = hv + jnp.dot(u, n2[...], preferred_element_type=_f32) + b2n[...]
    mu = jnp.mean(hn, axis=1, keepdims=True)
    ctr = hn - mu
    var = jnp.mean(ctr * ctr, axis=1, keepdims=True)
    hln = ctr * jax.lax.rsqrt(var + 1e-5) * g[...] + bb[...]
    h_o[...] = hln
    if coord:
        tacc = t0[...] + t1[...]
        deg = tacc[:, 3:4]
        invd = 1.0 / jnp.maximum(deg, 1.0)
        lane = lax.broadcasted_iota(jnp.int32, (1, 16), 1)
        xq_o[...] = xq[...] + jnp.where(lane < 3, tacc, 0.0) * invd
        a_o[...] = jnp.dot(hln, w1d[...], preferred_element_type=_f32)
        b_o[...] = jnp.dot(hln, w1s[...], preferred_element_type=_f32)


def _tc_node(coord, h, m0, m1, t0, t1, xq, n1a, n1b, b1n, n2, b2n, g, bb,
             w1d, w1s):
    grid = NP // BN
    in_specs = [_blk((BN, HH)), _blk((BN, HH)), _blk((BN, HH)),
                _blk((BN, 16)), _blk((BN, 16)), _blk((BN, 16))] + \
               [_full(w) for w in (n1a, n1b, b1n, n2, b2n, g, bb, w1d, w1s)]
    if coord:
        out_shape = (jax.ShapeDtypeStruct((NP, HH), _f32),
                     jax.ShapeDtypeStruct((NP, 16), _f32),
                     jax.ShapeDtypeStruct((NP, HH), _f32),
                     jax.ShapeDtypeStruct((NP, HH), _f32))
        out_specs = (_blk((BN, HH)), _blk((BN, 16)), _blk((BN, HH)),
                     _blk((BN, HH)))
    else:
        out_shape = jax.ShapeDtypeStruct((NP, HH), _f32)
        out_specs = _blk((BN, HH))
    return pl.pallas_call(
        functools.partial(_node_kernel_body, coord),
        grid=(grid,), in_specs=in_specs, out_specs=out_specs,
        out_shape=out_shape)(h, m0, m1, t0, t1, xq, n1a, n1b, b1n, n2, b2n,
                             g, bb, w1d, w1s)


def _init_kernel_body(nf, pos, wn, bn, w1d, w1s, h_o, xq_o, a_o, b_o):
    h = (jnp.dot(jnp.clip(nf[...], -100.0, 100.0), wn[...],
                 preferred_element_type=_f32) + bn[...])
    h_o[...] = h
    xq_o[...] = jnp.clip(pos[...], -500.0, 500.0)
    a_o[...] = jnp.dot(h, w1d[...], preferred_element_type=_f32)
    b_o[...] = jnp.dot(h, w1s[...], preferred_element_type=_f32)


def _tc_init(nfp, pos16, wn, bn, w1d, w1s):
    grid = NP // BN
    in_specs = [_blk((BN, 128)), _blk((BN, 16))] + \
               [_full(w) for w in (wn, bn, w1d, w1s)]
    out_shape = (jax.ShapeDtypeStruct((NP, HH), _f32),
                 jax.ShapeDtypeStruct((NP, 16), _f32),
                 jax.ShapeDtypeStruct((NP, HH), _f32),
                 jax.ShapeDtypeStruct((NP, HH), _f32))
    out_specs = (_blk((BN, HH)), _blk((BN, 16)), _blk((BN, HH)),
                 _blk((BN, HH)))
    return pl.pallas_call(
        _init_kernel_body, grid=(grid,), in_specs=in_specs,
        out_specs=out_specs, out_shape=out_shape)(nfp, pos16, wn, bn,
                                                  w1d, w1s)


def kernel(node_features, positions, edge_index, edge_attr, params):
    src = edge_index[0]
    dst = edge_index[1]
    dstp = jnp.pad(dst, (0, EP - EE), constant_values=NN)
    dst2 = dstp.reshape(NW * NCHUNK, CHUNK)
    src2 = jnp.pad(src, (0, EP - EE), constant_values=NN).reshape(
        NW * NCHUNK, CHUNK)
    eap = jnp.pad(edge_attr, ((0, EP - EE), (0, 0)))
    nfp = jnp.pad(node_features, ((0, NP - NN), (0, 0)))
    pos16 = jnp.pad(positions, ((0, NP - NN), (0, 13)))

    we = params["edge_embed"]["W"]
    be_ = params["edge_embed"]["b"][None]
    lw = []
    for lp in params["layers"]:
        w1 = lp["edge1"]["W"]
        lw.append(dict(
            w1d=w1[:HH], w1s=w1[HH:2 * HH], wd2=w1[2 * HH:2 * HH + 1],
            w1e=w1[2 * HH + 1:], b1=lp["edge1"]["b"][None],
            w2=lp["edge2"]["W"], b2=lp["edge2"]["b"][None],
            c1=lp["coord1"]["W"], c1b=lp["coord1"]["b"][None],
            c2r=lp["coord2"]["W"].T,
            n1a=lp["node1"]["W"][:HH], n1b=lp["node1"]["W"][HH:],
            b1n=lp["node1"]["b"][None], n2=lp["node2"]["W"],
            b2n=lp["node2"]["b"][None], g=lp["ln_g"][None],
            bb=lp["ln_b"][None]))

    h, xq, A, B = _tc_init(nfp, pos16, params["node_embed"]["W"],
                           params["node_embed"]["b"][None],
                           lw[0]["w1d"], lw[0]["w1s"])

    for i in range(NLAYER):
        w = lw[i]
        coord = i < NLAYER - 1
        GD, GS, XD, XS = _sc_gather(A, B, xq, dst2, src2)
        if coord:
            m2, T = _tc_edge(True, GD, GS, XD, XS, eap, we, w["w1e"],
                             w["b1"], be_, w["wd2"], w["w2"], w["b2"],
                             w["c1"], w["c1b"], w["c2r"])
            Magg, Tacc = _sc_scatter(m2, T, dstp)
            nx = lw[i + 1]
            h, xq, A, B = _tc_node(
                True, h, Magg[0], Magg[1], Tacc[0], Tacc[1], xq,
                w["n1a"], w["n1b"], w["b1n"], w["n2"], w["b2n"],
                w["g"], w["bb"], nx["w1d"], nx["w1s"])
        else:
            m2 = _tc_edge(False, GD, GS, XD, XS, eap, we, w["w1e"],
                          w["b1"], be_, w["wd2"], w["w2"], w["b2"],
                          w["c1"], w["c1b"], w["c2r"])
            Magg = _sc_scatter(m2, None, dstp)
            zt = jnp.zeros((NP, 16), _f32)
            h = _tc_node(False, h, Magg[0], Magg[1], zt, zt, xq,
                         w["n1a"], w["n1b"], w["b1n"], w["n2"], w["b2n"],
                         w["g"], w["bb"], w["w1d"], w["w1s"])
    return h[:NN]
